# Initial kernel scaffold; baseline (speedup 1.0000x reference)
import jax
import jax.numpy as jnp
from jax.experimental import pallas as pl


def kernel(input_items, masks, lengths, emb, W_level, w_mg, Wq, Wk, Wv, Wo, W1, W2, v_att, Wg, Wout, bout):
    raise NotImplementedError("write your pallas kernel here")


# speedup vs baseline: 5.5735x; 5.5735x over previous
import functools

import jax
import jax.numpy as jnp
from jax import lax
from jax.experimental import pallas as pl
from jax.experimental.pallas import tpu as pltpu
from jax.experimental.pallas import tpu_sc as plsc


@functools.lru_cache(maxsize=None)
def _make_sc_gather(V, D, N):
    info = plsc.get_sparse_core_info()
    NW = info.num_cores * info.num_subcores
    assert N % NW == 0
    n_per_w = N // NW
    CH = 1280
    assert n_per_w % CH == 0
    n_ch = n_per_w // CH
    mesh = plsc.VectorSubcoreMesh(core_axis_name="c", subcore_axis_name="s")

    @functools.partial(
        pl.kernel,
        mesh=mesh,
        compiler_params=pltpu.CompilerParams(use_tc_tiling_on_sc=False),
        out_type=jax.ShapeDtypeStruct((N, D), jnp.float32),
        scratch_types=[
            pltpu.VMEM((n_per_w,), jnp.int32),
            pltpu.VMEM((CH, D), jnp.float32),
            pltpu.VMEM((CH, D), jnp.float32),
            pltpu.SemaphoreType.DMA,
            pltpu.SemaphoreType.DMA,
        ],
    )
    def gather_k(table_hbm, idx_hbm, out_hbm, idx_v, buf0, buf1, sem0, sem1):
        wid = lax.axis_index("s") * info.num_cores + lax.axis_index("c")
        base = wid * n_per_w
        pltpu.sync_copy(idx_hbm.at[pl.ds(base, n_per_w)], idx_v)
        bufs = (buf0, buf1)
        sems = (sem0, sem1)
        cps = [None] * n_ch
        cps[0] = pltpu.async_copy(
            table_hbm.at[idx_v.at[pl.ds(0, CH)]], bufs[0], sems[0])
        for c in range(n_ch):
            if c + 1 < n_ch:
                cps[c + 1] = pltpu.async_copy(
                    table_hbm.at[idx_v.at[pl.ds((c + 1) * CH, CH)]],
                    bufs[(c + 1) % 2], sems[(c + 1) % 2])
            cps[c].wait()
            pltpu.sync_copy(bufs[c % 2], out_hbm.at[pl.ds(base + c * CH, CH)])

    return gather_k


def _tc_body(x_ref, masks_ref, len_ref, Wl_ref, wmg_ref, Wv_ref, Wo_ref,
             W1_ref, W2_ref, vatt_ref, Wg_ref, Wout_ref, bout_ref, out_ref):
    bB, L, D = x_ref.shape
    G = Wl_ref.shape[0]
    H = W1_ref.shape[1]
    f32 = jnp.float32

    x = x_ref[...]
    masks = masks_ref[...]
    lengths = len_ref[...]

    m3 = masks[:, :, None]
    denom = jnp.maximum(jnp.sum(masks, axis=1, keepdims=True), 1.0)
    initial = jnp.sum(x * m3, axis=1) / denom

    wmg = wmg_ref[...]
    levels = [jnp.dot(initial, Wl_ref[g], preferred_element_type=f32)
              for g in range(G)]
    scores = [jnp.sum(jnp.tanh(lv) * wmg, axis=1, keepdims=True)
              for lv in levels]
    smax = scores[0]
    for s in scores[1:]:
        smax = jnp.maximum(smax, s)
    exps = [jnp.exp(s - smax) for s in scores]
    ssum = exps[0]
    for e in exps[1:]:
        ssum = ssum + e
    mg = levels[0] * (exps[0] / ssum)
    for e, lv in zip(exps[1:], levels[1:]):
        mg = mg + lv * (e / ssum)

    c_vec = jnp.dot(jnp.dot(mg, Wv_ref[...], preferred_element_type=f32),
                    Wo_ref[...], preferred_element_type=f32)

    idx_last = jnp.clip(lengths - 1, 0, L - 1)
    lidx = lax.broadcasted_iota(jnp.int32, (bB, L), 1)
    onehot = (lidx == idx_last).astype(f32)
    x_last = jnp.sum(x * onehot[:, :, None], axis=1)

    xf = x.reshape(bB * L, D)
    xW1 = jnp.dot(xf, W1_ref[...], preferred_element_type=f32)
    xW1 = xW1.reshape(bB, L, H)
    vatt3 = vatt_ref[...].reshape(1, 1, H)
    neg = (1.0 - masks) * -1e9

    def readout(hW1_3, h3, h_last):
        a = jnp.dot(h_last, W2_ref[...], preferred_element_type=f32)
        e = jax.nn.sigmoid(hW1_3 + a[:, None, :])
        s = jnp.sum(e * vatt3, axis=2) + neg
        s = s - jnp.max(s, axis=1, keepdims=True)
        w = jnp.exp(s)
        w = w / jnp.sum(w, axis=1, keepdims=True)
        return jnp.sum(w[:, :, None] * h3, axis=1)

    out_g = readout(xW1, x, x_last)
    cW1 = jnp.dot(c_vec, W1_ref[...], preferred_element_type=f32)
    h3f = x + c_vec[:, None, :]
    out_f = readout(xW1 + cW1[:, None, :], h3f, x_last + c_vec)

    Wg = Wg_ref[...]
    gate = jax.nn.sigmoid(
        jnp.dot(out_g, Wg[:D], preferred_element_type=f32)
        + jnp.dot(out_f, Wg[D:], preferred_element_type=f32))
    final = gate * out_g + (1.0 - gate) * out_f
    out_ref[...] = (jnp.dot(final, Wout_ref[...], preferred_element_type=f32)
                    + bout_ref[...])


def _tc_call(x3, masks, len2, W_level, wmg2, Wv, Wo, W1, W2, vatt2, Wg,
             Wout, bout2):
    B, L, D = x3.shape
    G = W_level.shape[0]
    H = W1.shape[1]
    bB = 256
    grid = (B // bB,)

    def full(shape):
        return pl.BlockSpec(shape, lambda i: tuple(0 for _ in shape))

    return pl.pallas_call(
        _tc_body,
        grid=grid,
        in_specs=[
            pl.BlockSpec((bB, L, D), lambda i: (i, 0, 0)),
            pl.BlockSpec((bB, L), lambda i: (i, 0)),
            pl.BlockSpec((bB, 1), lambda i: (i, 0)),
            full((G, D, D)),
            full((1, D)),
            full((D, D)),
            full((D, D)),
            full((D, H)),
            full((D, H)),
            full((1, H)),
            full((2 * D, D)),
            full((D, D)),
            full((1, D)),
        ],
        out_specs=pl.BlockSpec((bB, D), lambda i: (i, 0)),
        out_shape=jax.ShapeDtypeStruct((B, D), jnp.float32),
        compiler_params=pltpu.CompilerParams(
            dimension_semantics=("parallel",)),
    )(x3, masks, len2, W_level, wmg2, Wv, Wo, W1, W2, vatt2, Wg, Wout, bout2)


def kernel(input_items, masks, lengths, emb, W_level, w_mg, Wq, Wk, Wv, Wo,
           W1, W2, v_att, Wg, Wout, bout):
    B, L = input_items.shape
    V, D = emb.shape
    H = W1.shape[1]
    flat_idx = input_items.reshape(-1).astype(jnp.int32)
    x_flat = _make_sc_gather(V, D, B * L)(emb, flat_idx)
    x3 = x_flat.reshape(B, L, D)
    return _tc_call(
        x3, masks, lengths.reshape(B, 1).astype(jnp.int32), W_level,
        w_mg.reshape(1, D), Wv, Wo, W1, W2, v_att.reshape(1, H), Wg, Wout,
        bout.reshape(1, D))


# speedup vs baseline: 5.7896x; 1.0388x over previous
import functools

import jax
import jax.numpy as jnp
from jax import lax
from jax.experimental import pallas as pl
from jax.experimental.pallas import tpu as pltpu
from jax.experimental.pallas import tpu_sc as plsc


@functools.lru_cache(maxsize=None)
def _make_sc_gather(V, D, N):
    info = plsc.get_sparse_core_info()
    NW = info.num_cores * info.num_subcores
    assert N % NW == 0
    n_per_w = N // NW
    CH = 1280
    assert n_per_w % CH == 0
    n_ch = n_per_w // CH
    mesh = plsc.VectorSubcoreMesh(core_axis_name="c", subcore_axis_name="s")

    @functools.partial(
        pl.kernel,
        mesh=mesh,
        compiler_params=pltpu.CompilerParams(use_tc_tiling_on_sc=False),
        out_type=jax.ShapeDtypeStruct((N, 128), jnp.float32),
        scratch_types=[
            pltpu.VMEM((n_per_w,), jnp.int32),
            pltpu.VMEM((CH, D), jnp.float32),
            pltpu.VMEM((CH, D), jnp.float32),
            pltpu.SemaphoreType.DMA,
            pltpu.SemaphoreType.DMA,
        ],
    )
    def gather_k(table_hbm, idx_hbm, out_hbm, idx_v, buf0, buf1, sem0, sem1):
        wid = lax.axis_index("s") * info.num_cores + lax.axis_index("c")
        base = wid * n_per_w
        pltpu.sync_copy(idx_hbm.at[pl.ds(base, n_per_w)], idx_v)
        bufs = (buf0, buf1)
        sems = (sem0, sem1)
        cps = [None] * n_ch
        cps[0] = pltpu.async_copy(
            table_hbm.at[idx_v.at[pl.ds(0, CH)]], bufs[0], sems[0])
        for c in range(n_ch):
            if c + 1 < n_ch:
                cps[c + 1] = pltpu.async_copy(
                    table_hbm.at[idx_v.at[pl.ds((c + 1) * CH, CH)]],
                    bufs[(c + 1) % 2], sems[(c + 1) % 2])
            cps[c].wait()
            pltpu.sync_copy(
                bufs[c % 2],
                out_hbm.at[pl.ds(base + c * CH, CH), pl.ds(0, D)])

    return gather_k


def _tc_body(x_ref, masks_ref, len_ref, Wlp_ref, wmgp_ref, Wvp_ref, Wop_ref,
             W1p_ref, W2p_ref, vattp_ref, Wgap_ref, Wgbp_ref, Woutp_ref,
             boutp_ref, out_ref, *, bB, L, D):
    G = Wlp_ref.shape[0]
    f32 = jnp.float32

    xr = x_ref[...]
    lane = lax.broadcasted_iota(jnp.int32, xr.shape, 1)
    xz = jnp.where(lane < D, xr, 0.0)
    x3 = xz.reshape(bB, L, 128)
    masks = masks_ref[...]
    lengths = len_ref[...]

    m3 = masks[:, :, None]
    denom = jnp.maximum(jnp.sum(masks, axis=1, keepdims=True), 1.0)
    initial = jnp.sum(x3 * m3, axis=1) / denom

    wmgp = wmgp_ref[...]
    levels = [jnp.dot(initial, Wlp_ref[g], preferred_element_type=f32)
              for g in range(G)]
    scores = [jnp.sum(jnp.tanh(lv) * wmgp, axis=1, keepdims=True)
              for lv in levels]
    smax = scores[0]
    for s in scores[1:]:
        smax = jnp.maximum(smax, s)
    exps = [jnp.exp(s - smax) for s in scores]
    ssum = exps[0]
    for e in exps[1:]:
        ssum = ssum + e
    mg = levels[0] * (exps[0] / ssum)
    for e, lv in zip(exps[1:], levels[1:]):
        mg = mg + lv * (e / ssum)

    c_vec = jnp.dot(jnp.dot(mg, Wvp_ref[...], preferred_element_type=f32),
                    Wop_ref[...], preferred_element_type=f32)

    idx_last = jnp.clip(lengths - 1, 0, L - 1)
    lidx = lax.broadcasted_iota(jnp.int32, (bB, L), 1)
    onehot = (lidx == idx_last).astype(f32)
    x_last = jnp.sum(x3 * onehot[:, :, None], axis=1)

    xW1 = jnp.dot(xz, W1p_ref[...], preferred_element_type=f32)
    xW1 = xW1.reshape(bB, L, 128)
    vatt3 = vattp_ref[...].reshape(1, 1, 128)
    neg = (1.0 - masks) * -1e9

    def readout(hW1_3, h_last):
        a = jnp.dot(h_last, W2p_ref[...], preferred_element_type=f32)
        e = jax.nn.sigmoid(hW1_3 + a[:, None, :])
        s = jnp.sum(e * vatt3, axis=2) + neg
        s = s - jnp.max(s, axis=1, keepdims=True)
        w = jnp.exp(s)
        w = w / jnp.sum(w, axis=1, keepdims=True)
        return jnp.sum(w[:, :, None] * x3, axis=1)

    out_g = readout(xW1, x_last)
    cW1 = jnp.dot(c_vec, W1p_ref[...], preferred_element_type=f32)
    out_f = readout(xW1 + cW1[:, None, :], x_last + c_vec) + c_vec

    gate = jax.nn.sigmoid(
        jnp.dot(out_g, Wgap_ref[...], preferred_element_type=f32)
        + jnp.dot(out_f, Wgbp_ref[...], preferred_element_type=f32))
    final = gate * out_g + (1.0 - gate) * out_f
    out_ref[...] = (jnp.dot(final, Woutp_ref[...], preferred_element_type=f32)
                    + boutp_ref[...])


def _tc_call(x128, masks, len2, W_level, w_mg, Wv, Wo, W1, W2, v_att, Wg,
             Wout, bout):
    B, L = masks.shape
    G, D = W_level.shape[0], W_level.shape[1]
    H = W1.shape[1]
    bB = 128
    grid = (B // bB,)
    rows = bB * L
    padD = 128 - D
    padH = 128 - H
    Wlp = jnp.pad(W_level, ((0, 0), (0, padD), (0, padD)))
    wmgp = jnp.pad(w_mg, (0, padD)).reshape(1, 128)
    Wvp = jnp.pad(Wv, ((0, padD), (0, padD)))
    Wop = jnp.pad(Wo, ((0, padD), (0, padD)))
    W1p = jnp.pad(W1, ((0, padD), (0, padH)))
    W2p = jnp.pad(W2, ((0, padD), (0, padH)))
    vattp = jnp.pad(v_att, (0, padH)).reshape(1, 128)
    Wgap = jnp.pad(Wg[:D], ((0, padD), (0, padD)))
    Wgbp = jnp.pad(Wg[D:], ((0, padD), (0, padD)))
    Woutp = jnp.pad(Wout, ((0, padD), (0, padD)))
    boutp = jnp.pad(bout, (0, padD)).reshape(1, 128)

    def full(shape):
        return pl.BlockSpec(shape, lambda i: tuple(0 for _ in shape))

    out128 = pl.pallas_call(
        functools.partial(_tc_body, bB=bB, L=L, D=D),
        grid=grid,
        in_specs=[
            pl.BlockSpec((rows, 128), lambda i: (i, 0)),
            pl.BlockSpec((bB, L), lambda i: (i, 0)),
            pl.BlockSpec((bB, 1), lambda i: (i, 0)),
            full((G, 128, 128)),
            full((1, 128)),
            full((128, 128)),
            full((128, 128)),
            full((128, 128)),
            full((128, 128)),
            full((1, 128)),
            full((128, 128)),
            full((128, 128)),
            full((128, 128)),
            full((1, 128)),
        ],
        out_specs=pl.BlockSpec((bB, 128), lambda i: (i, 0)),
        out_shape=jax.ShapeDtypeStruct((B, 128), jnp.float32),
        compiler_params=pltpu.CompilerParams(
            dimension_semantics=("parallel",)),
    )(x128, masks, len2, Wlp, wmgp, Wvp, Wop, W1p, W2p, vattp, Wgap, Wgbp,
      Woutp, boutp)
    return out128[:, :D]


def kernel(input_items, masks, lengths, emb, W_level, w_mg, Wq, Wk, Wv, Wo,
           W1, W2, v_att, Wg, Wout, bout):
    B, L = input_items.shape
    V, D = emb.shape
    flat_idx = input_items.reshape(-1).astype(jnp.int32)
    x128 = _make_sc_gather(V, D, B * L)(emb, flat_idx)
    return _tc_call(
        x128, masks, lengths.reshape(B, 1).astype(jnp.int32), W_level,
        w_mg, Wv, Wo, W1, W2, v_att, Wg, Wout, bout)
